# Initial kernel scaffold; baseline (speedup 1.0000x reference)
import jax
import jax.numpy as jnp
from jax.experimental import pallas as pl


def kernel(p_species, p_moves, p_ability, p_status, p_item, e_status, party_level, p_hp, p_lvl, p_att, p_defn, p_spe, p_spA, p_spD, p_pp, p_exp, e_hp, e_lvl, party_hp, inbattle, badge, hms, map_feat, species_emb, move_emb, ability_emb, status_emb, item_emb, e_status_emb, pW1, pb1, pW2, pb2, eW1, eb1, eW2, eb2, partyW, partyb, gW, gb, fW, fb):
    raise NotImplementedError("write your pallas kernel here")


# speedup vs baseline: 7.3601x; 7.3601x over previous
import jax
import jax.numpy as jnp
from jax.experimental import pallas as pl
from jax.experimental.pallas import tpu as pltpu


def _onehot_gather(idx_col, table):
    n = idx_col.shape[0]
    v = table.shape[0]
    iota = jax.lax.broadcasted_iota(jnp.int32, (n, v), 1)
    oh = (iota == idx_col[:, None]).astype(jnp.float32)
    return jax.lax.dot_general(
        oh, table, (((1,), (0,)), ((), ())),
        preferred_element_type=jnp.float32)


def _body(sp_ref, mv_ref, ab_ref, st_ref, it_ref, est_ref, plvl_ref,
          hp_ref, lvl_ref, att_ref, defn_ref, spe_ref, spA_ref, spD_ref,
          pp_ref, exp_ref, ehp_ref, elvl_ref, phl_ref, inb_ref, badge_ref,
          hms_ref, map_ref, semb_ref, memb_ref, aemb_ref, stemb_ref,
          iemb_ref, esemb_ref, pW1_ref, pb1_ref, pW2_ref, pb2_ref,
          eW1_ref, eb1_ref, eW2_ref, eb2_ref, paW_ref, pab_ref,
          gW_ref, gb_ref, fW_ref, fb_ref, out_ref):
    n = sp_ref.shape[0]
    semb = semb_ref[...]
    memb = memb_ref[...]
    aemb = aemb_ref[...]
    stemb = stemb_ref[...]
    iemb = iemb_ref[...]
    esemb = esemb_ref[...]
    pW1 = pW1_ref[...]
    pW2 = pW2_ref[...]
    eW1 = eW1_ref[...]
    eW2 = eW2_ref[...]
    pb1 = pb1_ref[...]
    pb2 = pb2_ref[...]
    eb1 = eb1_ref[...]
    eb2 = eb2_ref[...]

    player_acc = jnp.zeros((n, 128), jnp.float32)
    enemy_acc = jnp.zeros((n, 128), jnp.float32)
    for p in range(6):
        sp_e = _onehot_gather(sp_ref[:, p], semb)
        mv_e = [_onehot_gather(mv_ref[:, 4 * p + j], memb) for j in range(4)]
        ab_e = _onehot_gather(ab_ref[:, p], aemb)
        st_e = _onehot_gather(st_ref[:, p], stemb)
        it_e = _onehot_gather(it_ref[:, p], iemb)
        pp_mean = jnp.mean(pp_ref[:, 4 * p:4 * p + 4], axis=1, keepdims=True)
        stats = jnp.concatenate([
            hp_ref[:, p:p + 1], lvl_ref[:, p:p + 1] / 100.0,
            att_ref[:, p:p + 1], defn_ref[:, p:p + 1], spe_ref[:, p:p + 1],
            spA_ref[:, p:p + 1], spD_ref[:, p:p + 1], pp_mean,
            exp_ref[:, p:p + 1]], axis=1)
        px = jnp.concatenate(
            [sp_e, mv_e[0], mv_e[1], mv_e[2], mv_e[3], ab_e, st_e, it_e,
             stats], axis=1)
        h1 = jax.nn.relu(jnp.dot(px, pW1,
                                 preferred_element_type=jnp.float32) + pb1)
        h2 = jnp.dot(h1, pW2, preferred_element_type=jnp.float32) + pb2
        player_acc = player_acc + h2

        es_e = _onehot_gather(est_ref[:, p], esemb)
        ex = jnp.concatenate(
            [es_e, ehp_ref[:, p:p + 1], elvl_ref[:, p:p + 1] / 100.0], axis=1)
        g1 = jax.nn.relu(jnp.dot(ex, eW1,
                                 preferred_element_type=jnp.float32) + eb1)
        g2 = jnp.dot(g1, eW2, preferred_element_type=jnp.float32) + eb2
        enemy_acc = enemy_acc + g2

    player = player_acc * (1.0 / 6.0)
    enemy = enemy_acc * (1.0 / 6.0)

    php = jnp.mean(phl_ref[...], axis=1, keepdims=True)
    plv = jnp.mean(plvl_ref[...].astype(jnp.float32) / 100.0, axis=1,
                   keepdims=True)
    party = (php * paW_ref[0:1, :] + plv * paW_ref[1:2, :] + pab_ref[...])

    g_in = jnp.concatenate([inb_ref[...], badge_ref[...], hms_ref[...]],
                           axis=1)
    gv = jax.nn.relu(jnp.dot(g_in, gW_ref[...],
                             preferred_element_type=jnp.float32) + gb_ref[...])

    x = jnp.concatenate([player, enemy, party, map_ref[...], gv], axis=1)
    out = jax.nn.relu(jnp.dot(x, fW_ref[...],
                              preferred_element_type=jnp.float32) + fb_ref[...])
    out_ref[...] = out


def kernel(p_species, p_moves, p_ability, p_status, p_item, e_status,
           party_level, p_hp, p_lvl, p_att, p_defn, p_spe, p_spA, p_spD,
           p_pp, p_exp, e_hp, e_lvl, party_hp, inbattle, badge, hms,
           map_feat, species_emb, move_emb, ability_emb, status_emb,
           item_emb, e_status_emb, pW1, pb1, pW2, pb2, eW1, eb1, eW2, eb2,
           partyW, partyb, gW, gb, fW, fb):
    b = p_species.shape[0]
    n = min(512, b)
    grid = (b // n,)

    mv = p_moves.reshape(b, 24).astype(jnp.int32)
    pp = p_pp.reshape(b, 24)

    def bspec(k):
        return pl.BlockSpec((n, k), lambda i: (i, 0))

    def wspec(shape):
        nd = len(shape)
        return pl.BlockSpec(shape, lambda i: (0,) * nd)

    batch_in = [
        (p_species.astype(jnp.int32), 6), (mv, 24),
        (p_ability.astype(jnp.int32), 6), (p_status.astype(jnp.int32), 6),
        (p_item.astype(jnp.int32), 6), (e_status.astype(jnp.int32), 6),
        (party_level.astype(jnp.int32), 6),
        (p_hp, 6), (p_lvl, 6), (p_att, 6), (p_defn, 6), (p_spe, 6),
        (p_spA, 6), (p_spD, 6), (pp, 24), (p_exp, 6), (e_hp, 6), (e_lvl, 6),
        (party_hp, 6), (inbattle, 1), (badge, 8), (hms, 8), (map_feat, 55),
    ]
    weights = [species_emb, move_emb, ability_emb, status_emb, item_emb,
               e_status_emb, pW1, pb1.reshape(1, 128), pW2,
               pb2.reshape(1, 128), eW1, eb1.reshape(1, 128), eW2,
               eb2.reshape(1, 128), partyW, partyb.reshape(1, 128), gW,
               gb.reshape(1, 32), fW, fb.reshape(1, 256)]

    in_specs = [bspec(k) for _, k in batch_in] + [wspec(w.shape)
                                                 for w in weights]
    args = [a for a, _ in batch_in] + weights

    return pl.pallas_call(
        _body,
        grid=grid,
        in_specs=in_specs,
        out_specs=pl.BlockSpec((n, 256), lambda i: (i, 0)),
        out_shape=jax.ShapeDtypeStruct((b, 256), jnp.float32),
        compiler_params=pltpu.CompilerParams(
            dimension_semantics=("parallel",)),
    )(*args)
